# Initial kernel scaffold; baseline (speedup 1.0000x reference)
import jax
import jax.numpy as jnp
from jax.experimental import pallas as pl


def kernel(p3_o2o, p4_o2o, p5_o2o, p3_o2m, p4_o2m, p5_o2m, gt_bboxes, gt_labels, ac3, ac4, ac5):
    raise NotImplementedError("write your pallas kernel here")


# speedup vs baseline: 47.0136x; 47.0136x over previous
import jax
import jax.numpy as jnp
import numpy as np
from jax.experimental import pallas as pl
from jax.experimental.pallas import tpu as pltpu

_NC = 80
_RL = 16
_TOPK = 10
_EPS = 1e-9
_G = 32
_REG_HI = 15 - 0.01


def _atan_pos(x):
    for _ in range(3):
        x = x / (1.0 + jnp.sqrt(1.0 + x * x))
    x2 = x * x
    return 8.0 * x * (1.0 + x2 * (-1.0 / 3.0 + x2 * (0.2 + x2 * (-1.0 / 7.0))))


def _loss_kernel(x_ref, aux_ref, gtb_ref, gtl_ref, out_ref,
                 box_ref, met_ref, iou_ref, xlab_ref, wk_ref):
    f32 = jnp.float32
    N = x_ref.shape[2]
    cx = aux_ref[0:1, :]
    cy = aux_ref[1:2, :]
    sv = aux_ref[2:3, :]

    jw = jax.lax.broadcasted_iota(jnp.int32, (_RL, 1), 0).astype(f32)
    for s in range(4):
        rows = x_ref[0, _RL * s:_RL * (s + 1), :]
        m = jnp.max(rows, axis=0, keepdims=True)
        e = jnp.exp(rows - m)
        den = jnp.sum(e, axis=0, keepdims=True)
        num = jnp.sum(e * jw, axis=0, keepdims=True)
        ev = num / den * sv
        box_ref[4 + s:5 + s, :] = m + jnp.log(den)
        if s == 0:
            box_ref[0:1, :] = cx - ev
        elif s == 1:
            box_ref[1:2, :] = cy - ev
        elif s == 2:
            box_ref[2:3, :] = cx + ev
        else:
            box_ref[3:4, :] = cy + ev
    bx1 = box_ref[0:1, :]
    by1 = box_ref[1:2, :]
    bx2 = box_ref[2:3, :]
    by2 = box_ref[3:4, :]
    a1v = jnp.maximum(bx2 - bx1, 0.0) * jnp.maximum(by2 - by1, 0.0)
    box_ref[8:9, :] = a1v

    bacc = jnp.zeros((1, N), f32)
    for t in range(4 * _RL, 4 * _RL + _NC, _RL):
        xx = x_ref[0, t:t + _RL, :]
        bacc = bacc + jnp.sum(
            jnp.maximum(xx, 0.0) + jnp.log(1.0 + jnp.exp(-jnp.abs(xx))),
            axis=0, keepdims=True)

    def g_body(g, carry):
        c = gtl_ref[0, 0, g]
        xr = x_ref[0, pl.ds(4 * _RL + c, 1), :]
        xlab_ref[pl.ds(g, 1), :] = xr
        sc = 1.0 / (1.0 + jnp.exp(-xr))
        gx1 = gtb_ref[0, g, 0]
        gy1 = gtb_ref[0, g, 1]
        gx2 = gtb_ref[0, g, 2]
        gy2 = gtb_ref[0, g, 3]
        xi1 = jnp.maximum(bx1, gx1)
        yi1 = jnp.maximum(by1, gy1)
        xi2 = jnp.minimum(bx2, gx2)
        yi2 = jnp.minimum(by2, gy2)
        inter = jnp.maximum(xi2 - xi1, 0.0) * jnp.maximum(yi2 - yi1, 0.0)
        a2 = jnp.maximum(gx2 - gx1, 0.0) * jnp.maximum(gy2 - gy1, 0.0)
        iou = jnp.maximum(inter / (a1v + a2 - inter + _EPS), 0.0)
        inbox = (cx > gx1) & (cy > gy1) & (cx < gx2) & (cy < gy2)
        iou2 = iou * iou
        align = jnp.sqrt(sc) * (iou2 * iou2 * iou2)
        met_ref[pl.ds(g, 1), :] = jnp.where(inbox, align, 0.0)
        iou_ref[pl.ds(g, 1), :] = iou
        return carry

    jax.lax.fori_loop(0, _G, g_body, 0, unroll=False)

    wk_ref[...] = met_ref[...]
    lane = jax.lax.broadcasted_iota(jnp.int32, (_G, N), 1)

    def k_body(k, carry):
        mt = wk_ref[...]
        m = jnp.max(mt, axis=1, keepdims=True)
        idx = jnp.min(jnp.where(mt >= m, lane, N), axis=1, keepdims=True)
        wk_ref[...] = jnp.where(lane == idx, -jnp.inf, mt)
        return carry

    jax.lax.fori_loop(0, _TOPK - 1, k_body, 0, unroll=False)
    thr = jnp.max(wk_ref[...], axis=1, keepdims=True)

    met = met_ref[...]
    iouv = iou_ref[...]
    tmf = ((met >= jnp.maximum(thr, _EPS)) & (met > 0.0)).astype(f32)
    nposf = jnp.sum(tmf, axis=0, keepdims=True)
    iom = jnp.where(tmf > 0.0, iouv, -1.0)
    bestv = jnp.max(iom, axis=0, keepdims=True)
    gio = jax.lax.broadcasted_iota(jnp.int32, (_G, N), 0)
    best = jnp.min(jnp.where(iom >= bestv, gio, _G), axis=0, keepdims=True)
    maskf = jnp.where(nposf > 1.0, (gio == best).astype(f32), tmf)
    fgf = (nposf > 0.0).astype(f32)
    matched = jnp.min(jnp.where(maskf > 0.0, gio, _G), axis=0, keepdims=True)
    matched = jnp.where(nposf > 0.0, matched, 0)
    mpa = jnp.max(met * maskf, axis=1, keepdims=True)
    mpi = jnp.max(iouv * maskf, axis=1, keepdims=True)
    normv = mpi / (mpa + _EPS)
    eqf = (gio == matched).astype(f32)
    pa = jnp.sum(eqf * met, axis=0, keepdims=True)
    na = jnp.sum(eqf * normv, axis=0, keepdims=True)
    xg = jnp.sum(eqf * xlab_ref[...], axis=0, keepdims=True)
    tval = pa * na * fgf

    box_ref[9:13, :] = jnp.zeros((4, N), f32)

    def tb_body(g, carry):
        eq = (matched == g).astype(f32)
        box_ref[9:10, :] += eq * gtb_ref[0, g, 0]
        box_ref[10:11, :] += eq * gtb_ref[0, g, 1]
        box_ref[11:12, :] += eq * gtb_ref[0, g, 2]
        box_ref[12:13, :] += eq * gtb_ref[0, g, 3]
        return carry

    jax.lax.fori_loop(0, _G, tb_body, 0, unroll=False)
    tx1 = box_ref[9:10, :]
    ty1 = box_ref[10:11, :]
    tx2 = box_ref[11:12, :]
    ty2 = box_ref[12:13, :]

    xi1 = jnp.maximum(bx1, tx1)
    yi1 = jnp.maximum(by1, ty1)
    xi2 = jnp.minimum(bx2, tx2)
    yi2 = jnp.minimum(by2, ty2)
    inter = jnp.maximum(xi2 - xi1, 0.0) * jnp.maximum(yi2 - yi1, 0.0)
    w1 = jnp.maximum(bx2 - bx1, 0.0)
    h1 = jnp.maximum(by2 - by1, 0.0)
    w2 = jnp.maximum(tx2 - tx1, 0.0)
    h2 = jnp.maximum(ty2 - ty1, 0.0)
    union = w1 * h1 + w2 * h2 - inter + _EPS
    iou = inter / union
    cw = jnp.maximum(bx2, tx2) - jnp.minimum(bx1, tx1)
    ch = jnp.maximum(by2, ty2) - jnp.minimum(by1, ty1)
    c2 = cw * cw + ch * ch + _EPS
    rho2 = ((bx1 + bx2 - tx1 - tx2) ** 2 + (by1 + by2 - ty1 - ty2) ** 2) / 4.0
    dv = _atan_pos(w2 / (h2 + _EPS)) - _atan_pos(w1 / (h1 + _EPS))
    v = (4.0 / (np.pi ** 2)) * dv * dv
    alpha = v / (v - iou + 1.0 + _EPS)
    ciou = iou - rho2 / c2 - alpha * v
    iou_num = jnp.sum((1.0 - ciou) * tval)

    dfl_acc = jnp.zeros((1, N), f32)
    for s in range(4):
        if s == 0:
            tdist = (cx - tx1) / sv
        elif s == 1:
            tdist = (cy - ty1) / sv
        elif s == 2:
            tdist = (tx2 - cx) / sv
        else:
            tdist = (ty2 - cy) / sv
        tdist = jnp.clip(tdist, 0.0, _REG_HI)
        gsum = jnp.zeros((1, N), f32)
        for j in range(_RL):
            wj = jnp.maximum(1.0 - jnp.abs(tdist - float(j)), 0.0)
            gsum = gsum + wj * x_ref[0, _RL * s + j:_RL * s + j + 1, :]
        dfl_acc = dfl_acc + (box_ref[4 + s:5 + s, :] - gsum)
    dfl_num = jnp.sum(dfl_acc * 0.25 * tval)

    bce_part = jnp.sum(bacc) - jnp.sum(xg * tval)
    ts_part = jnp.sum(tval)
    li = jax.lax.broadcasted_iota(jnp.int32, (1, 8, 128), 2)
    vec = jnp.where(li == 0, bce_part,
                    jnp.where(li == 1, ts_part,
                              jnp.where(li == 2, iou_num,
                                        jnp.where(li == 3, dfl_num, 0.0))))
    out_ref[...] = vec


def kernel(p3_o2o, p4_o2o, p5_o2o, p3_o2m, p4_o2m, p5_o2m,
           gt_bboxes, gt_labels, ac3, ac4, ac5):
    levels = [p3_o2o, p4_o2o, p5_o2o, p3_o2m, p4_o2m, p5_o2m]
    centers = [ac3, ac4, ac5, ac3, ac4, ac5]
    strides = [8, 16, 32, 8, 16, 32]
    B = p3_o2o.shape[0]
    X = jnp.concatenate(levels, axis=1)
    N = X.shape[1]
    Xt = jnp.transpose(X, (0, 2, 1))
    ccat = jnp.concatenate(centers, axis=0)
    spos = jnp.concatenate(
        [jnp.full((c.shape[0],), float(s), jnp.float32)
         for c, s in zip(centers, strides)], axis=0)
    aux = jnp.zeros((8, N), jnp.float32)
    aux = aux.at[0].set(ccat[:, 0]).at[1].set(ccat[:, 1]).at[2].set(spos)
    gtl = gt_labels.astype(jnp.int32).reshape(B, 1, _G)

    parts = pl.pallas_call(
        _loss_kernel,
        grid=(B,),
        in_specs=[
            pl.BlockSpec((1, 144, N), lambda b: (b, 0, 0)),
            pl.BlockSpec((8, N), lambda b: (0, 0)),
            pl.BlockSpec((1, _G, 4), lambda b: (b, 0, 0),
                         memory_space=pltpu.SMEM),
            pl.BlockSpec((1, 1, _G), lambda b: (b, 0, 0),
                         memory_space=pltpu.SMEM),
        ],
        out_specs=pl.BlockSpec((1, 8, 128), lambda b: (b, 0, 0)),
        out_shape=jax.ShapeDtypeStruct((B, 8, 128), jnp.float32),
        scratch_shapes=[
            pltpu.VMEM((16, N), jnp.float32),
            pltpu.VMEM((_G, N), jnp.float32),
            pltpu.VMEM((_G, N), jnp.float32),
            pltpu.VMEM((_G, N), jnp.float32),
            pltpu.VMEM((_G, N), jnp.float32),
        ],
        compiler_params=pltpu.CompilerParams(
            dimension_semantics=(pltpu.GridDimensionSemantics.PARALLEL,),
            vmem_limit_bytes=100 * 1024 * 1024,
        ),
    )(Xt, aux, gt_bboxes, gtl)

    s = jnp.sum(parts[:, 0, 0:4], axis=0)
    tsum = jnp.maximum(s[1], 1.0)
    cls_loss = s[0] / tsum
    iou_loss = s[2] / tsum
    dfl_loss = s[3] / tsum
    total = cls_loss + 2.5 * iou_loss + dfl_loss
    return jnp.stack([total, cls_loss, iou_loss, dfl_loss])
